# baseline (speedup 1.0000x reference)
import jax
import jax.numpy as jnp
from jax import lax
from jax.experimental import pallas as pl
from jax.experimental.pallas import tpu as pltpu
from jax.experimental.pallas import tpu_sc as plsc

_INFO = plsc.get_sparse_core_info()
_NC, _NS = _INFO.num_cores, _INFO.num_subcores
_NW = _NC * _NS

_NJ = 50
_NI = 16384
_D = 64
_NIB = _NI // 128
_NU = _NJ * _NIB
_UPW = _NU // _NW
_NPB = 4
_NTB = 2


def _body(table_hbm, idx_hbm, out_hbm, idx_v, pidx_v, pair_v, t_v,
          g0, g1, g2, g3, o0, o1, so0, so1, isem):
    wid = lax.axis_index("s") * _NC + lax.axis_index("c")
    u0 = wid * _UPW
    gsems = (g0, g1, g2, g3)
    osems = (o0, o1)
    sosems = (so0, so1)

    iota16 = lax.iota(jnp.int32, 16)
    rowv = [iota16 + (lb * 16) for lb in range(8)]

    def idx_dma(u, bi):
        uu = jnp.minimum(u0 + u, _NU - 1)
        return pltpu.async_copy(idx_hbm.at[uu // _NIB, uu % _NIB],
                                idx_v.at[bi], isem)

    def wait_idx(bi):
        pltpu.make_async_copy(idx_hbm.at[0, 0], idx_v.at[bi], isem).wait()

    def compute_pairidx(bi):
        for lb in range(8):
            iv = idx_v[bi, pl.ds(lb * 16, 16)]
            pidx_v[bi, pl.ds(lb * 16, 16)] = lax.shift_right_logical(iv, 1)

    def fire_gather(bi):
        pltpu.async_copy(table_hbm.at[pidx_v.at[bi]], pair_v.at[bi],
                         gsems[bi])

    def wait_gather(bi):
        pltpu.make_async_copy(table_hbm.at[pidx_v.at[bi]], pair_v.at[bi],
                              gsems[bi]).wait()

    def transpose(bi, ti):
        par64 = []
        for lb in range(8):
            iv = idx_v[bi, pl.ds(lb * 16, 16)]
            par64.append(lax.shift_left(iv & 1, 6))

        steps = [(ds, lb) for ds in range(8) for lb in range(8)]
        lat = 24

        @pl.loop(0, 8)
        def _d(dblk):
            dd = dblk * 8
            vals = []
            for s, (ds, lb) in enumerate(steps):
                colv = par64[lb] + (dd + ds)
                vals.append(
                    plsc.load_gather(pair_v.at[bi], [rowv[lb], colv]))
                if s >= lat:
                    ds0, lb0 = steps[s - lat]
                    t_v[ti, dblk, ds0, pl.ds(lb0 * 16, 16)] = vals[s - lat]
            for s in range(64 - lat, 64):
                ds0, lb0 = steps[s]
                t_v[ti, dblk, ds0, pl.ds(lb0 * 16, 16)] = vals[s]

    def store_tiles(u, ti):
        uu = u0 + u
        pltpu.async_copy(t_v.at[ti],
                         out_hbm.at[uu // _NIB, :, uu % _NIB],
                         osems[ti])

    def wait_store(ti):
        pltpu.make_async_copy(t_v.at[ti], out_hbm.at[0, :, 0],
                              osems[ti]).wait()

    def substep(u, pb, tb, do_ws):
        pb2 = (pb + 2) % _NPB
        pb3 = (pb + 3) % _NPB
        wait_idx(pb2)
        compute_pairidx(pb2)
        fire_gather(pb2)
        idx_dma(u + 3, pb3)
        wait_gather(pb)
        if do_ws:
            wait_store(tb)
        transpose(pb, tb)
        store_tiles(u, tb)

    pltpu.sync_copy(idx_hbm.at[u0 // _NIB, u0 % _NIB], idx_v.at[0])
    u1 = u0 + 1
    pltpu.sync_copy(idx_hbm.at[u1 // _NIB, u1 % _NIB], idx_v.at[1])
    compute_pairidx(0)
    fire_gather(0)
    compute_pairidx(1)
    fire_gather(1)
    idx_dma(2, 2)
    substep(0, 0, 0, False)
    substep(1, 1, 1, False)

    @pl.loop(0, (_UPW - 4) // _NPB)
    def _p(p):
        for s in range(_NPB):
            u = 2 + p * _NPB + s
            substep(u, (2 + s) % _NPB, s % _NTB, True)

    wait_gather(198 % _NPB)
    wait_store(0)
    transpose(198 % _NPB, 0)
    store_tiles(198, 0)
    wait_gather(199 % _NPB)
    wait_store(1)
    transpose(199 % _NPB, 1)
    store_tiles(199, 1)
    wait_idx((199 + 3) % _NPB)
    wait_store(0)
    wait_store(1)


@jax.jit
def _run(idx3, table2):
    mesh = plsc.VectorSubcoreMesh(core_axis_name="c", subcore_axis_name="s")
    k = pl.kernel(
        _body,
        out_type=jax.ShapeDtypeStruct((_NJ, 8, _NIB, 8, 128), jnp.float32),
        mesh=mesh,
        scratch_types=[
            pltpu.VMEM((_NPB, 128), jnp.int32),
            pltpu.VMEM((_NPB, 128), jnp.int32),
            pltpu.VMEM((_NPB, 128, 128), jnp.float32),
            pltpu.VMEM((_NTB, 8, 8, 128), jnp.float32),
        ] + [pltpu.SemaphoreType.DMA] * 9,
        compiler_params=pltpu.CompilerParams(use_tc_tiling_on_sc=True,
                                             needs_layout_passes=False),
    )
    return k(table2, idx3)


def kernel(iid, table):
    table2 = table.reshape(500000, 128)
    idx3 = iid.T.reshape(_NJ, _NIB, 128).astype(jnp.int32)
    out5 = _run(idx3, table2)
    return out5.transpose(2, 4, 0, 1, 3).reshape(_NI, _NJ, _D)


# speedup vs baseline: 1.8569x; 1.8569x over previous
import jax
import jax.numpy as jnp
from jax import lax
from jax.experimental import pallas as pl
from jax.experimental.pallas import tpu as pltpu
from jax.experimental.pallas import tpu_sc as plsc

_INFO = plsc.get_sparse_core_info()
_NC, _NS = _INFO.num_cores, _INFO.num_subcores
_NW = _NC * _NS

_NJ = 50
_NI = 16384
_D = 64
_NIB = _NI // 128
_NU = _NJ * _NIB
_UPW = _NU // _NW
_NPB = 4
_NTB = 2


def _body(table_hbm, idx_hbm, out_hbm, idx_v, pidx_v, pair_v, t_v,
          g0, g1, g2, g3, o0, o1, so0, so1, isem):
    wid = lax.axis_index("s") * _NC + lax.axis_index("c")
    u0 = wid * _UPW
    gsems = (g0, g1, g2, g3)
    osems = (o0, o1)
    sosems = (so0, so1)

    iota16 = lax.iota(jnp.int32, 16)
    rowv = [iota16 + (lb * 16) for lb in range(8)]

    def idx_dma(u, bi):
        uu = jnp.minimum(u0 + u, _NU - 1)
        return pltpu.async_copy(idx_hbm.at[uu // _NIB, uu % _NIB],
                                idx_v.at[bi], isem)

    def wait_idx(bi):
        pltpu.make_async_copy(idx_hbm.at[0, 0], idx_v.at[bi], isem).wait()

    def compute_pairidx(bi):
        for lb in range(8):
            iv = idx_v[bi, pl.ds(lb * 16, 16)]
            pidx_v[bi, pl.ds(lb * 16, 16)] = lax.shift_right_logical(iv, 1)

    def fire_gather(bi):
        pltpu.async_copy(table_hbm.at[pidx_v.at[bi]], pair_v.at[bi],
                         gsems[bi])

    def wait_gather(bi):
        pltpu.make_async_copy(table_hbm.at[pidx_v.at[bi]], pair_v.at[bi],
                              gsems[bi]).wait()

    def transpose(bi, ti):
        par64 = []
        for lb in range(8):
            iv = idx_v[bi, pl.ds(lb * 16, 16)]
            par64.append(lax.shift_left(iv & 1, 6))

        wrap = [(iota16 + m) & 15 for m in range(16)]
        lat = 8

        @pl.loop(0, 4)
        def _db(db):
            d0 = db * 16
            pend = []
            for m in range(16):
                ddv = wrap[m] + d0
                dblkv = lax.shift_right_logical(ddv, 3)
                dsv = ddv & 7
                for lb in range(8):
                    colv = par64[lb] + ddv
                    val = plsc.load_gather(pair_v.at[bi],
                                           [rowv[lb], colv])
                    pend.append((dblkv, dsv, rowv[lb], val))
                    if len(pend) > lat:
                        a, b, c, v = pend.pop(0)
                        plsc.store_scatter(t_v.at[ti], [a, b, c], v)
            for a, b, c, v in pend:
                plsc.store_scatter(t_v.at[ti], [a, b, c], v)

    def store_tiles(u, ti):
        uu = u0 + u
        pltpu.async_copy(t_v.at[ti],
                         out_hbm.at[uu // _NIB, :, uu % _NIB],
                         osems[ti])

    def wait_store(ti):
        pltpu.make_async_copy(t_v.at[ti], out_hbm.at[0, :, 0],
                              osems[ti]).wait()

    def substep(u, pb, tb, do_ws):
        pb2 = (pb + 2) % _NPB
        pb3 = (pb + 3) % _NPB
        wait_idx(pb2)
        compute_pairidx(pb2)
        fire_gather(pb2)
        idx_dma(u + 3, pb3)
        wait_gather(pb)
        if do_ws:
            wait_store(tb)
        transpose(pb, tb)
        store_tiles(u, tb)

    pltpu.sync_copy(idx_hbm.at[u0 // _NIB, u0 % _NIB], idx_v.at[0])
    u1 = u0 + 1
    pltpu.sync_copy(idx_hbm.at[u1 // _NIB, u1 % _NIB], idx_v.at[1])
    compute_pairidx(0)
    fire_gather(0)
    compute_pairidx(1)
    fire_gather(1)
    idx_dma(2, 2)
    substep(0, 0, 0, False)
    substep(1, 1, 1, False)

    @pl.loop(0, (_UPW - 4) // _NPB)
    def _p(p):
        for s in range(_NPB):
            u = 2 + p * _NPB + s
            substep(u, (2 + s) % _NPB, s % _NTB, True)

    wait_gather(198 % _NPB)
    wait_store(0)
    transpose(198 % _NPB, 0)
    store_tiles(198, 0)
    wait_gather(199 % _NPB)
    wait_store(1)
    transpose(199 % _NPB, 1)
    store_tiles(199, 1)
    wait_idx((199 + 3) % _NPB)
    wait_store(0)
    wait_store(1)


@jax.jit
def _run(idx3, table2):
    mesh = plsc.VectorSubcoreMesh(core_axis_name="c", subcore_axis_name="s")
    k = pl.kernel(
        _body,
        out_type=jax.ShapeDtypeStruct((_NJ, 8, _NIB, 8, 128), jnp.float32),
        mesh=mesh,
        scratch_types=[
            pltpu.VMEM((_NPB, 128), jnp.int32),
            pltpu.VMEM((_NPB, 128), jnp.int32),
            pltpu.VMEM((_NPB, 128, 128), jnp.float32),
            pltpu.VMEM((_NTB, 8, 8, 128), jnp.float32),
        ] + [pltpu.SemaphoreType.DMA] * 9,
        compiler_params=pltpu.CompilerParams(use_tc_tiling_on_sc=True,
                                             needs_layout_passes=False),
    )
    return k(table2, idx3)


def kernel(iid, table):
    table2 = table.reshape(500000, 128)
    idx3 = iid.T.reshape(_NJ, _NIB, 128).astype(jnp.int32)
    out5 = _run(idx3, table2)
    return out5.transpose(2, 4, 0, 1, 3).reshape(_NI, _NJ, _D)


# speedup vs baseline: 2.8800x; 1.5509x over previous
import jax
import jax.numpy as jnp
from jax import lax
from jax.experimental import pallas as pl
from jax.experimental.pallas import tpu as pltpu
from jax.experimental.pallas import tpu_sc as plsc

_INFO = plsc.get_sparse_core_info()
_NC, _NS = _INFO.num_cores, _INFO.num_subcores
_NW = _NC * _NS

_NJ = 50
_NI = 16384
_D = 64
_NIB = _NI // 128
_NU = _NJ * _NIB
_UPW = _NU // _NW
_NPB = 4
_NTB = 2


def _body(table_hbm, idx_hbm, out_hbm, idx_v, pidx_v, pair_v, t_v,
          g0, g1, g2, g3, o0, o1, so0, so1, isem):
    wid = lax.axis_index("s") * _NC + lax.axis_index("c")
    u0 = wid * _UPW
    gsems = (g0, g1, g2, g3)
    osems = (o0, o1)
    sosems = (so0, so1)

    iota16 = lax.iota(jnp.int32, 16)
    rowv = [iota16 + (lb * 16) for lb in range(8)]

    def idx_dma(u, bi):
        uu = jnp.minimum(u0 + u, _NU - 1)
        return pltpu.async_copy(idx_hbm.at[uu // _NIB, uu % _NIB],
                                idx_v.at[bi], isem)

    def wait_idx(bi):
        pltpu.make_async_copy(idx_hbm.at[0, 0], idx_v.at[bi], isem).wait()

    def compute_pairidx(bi):
        for lb in range(8):
            iv = idx_v[bi, pl.ds(lb * 16, 16)]
            pidx_v[bi, pl.ds(lb * 16, 16)] = lax.shift_right_logical(iv, 1)

    def fire_gather(bi):
        pltpu.async_copy(table_hbm.at[pidx_v.at[bi]], pair_v.at[bi],
                         gsems[bi])

    def wait_gather(bi):
        pltpu.make_async_copy(table_hbm.at[pidx_v.at[bi]], pair_v.at[bi],
                              gsems[bi]).wait()

    def transpose(bi, ti):
        par64 = []
        for lb in range(8):
            iv = idx_v[bi, pl.ds(lb * 16, 16)]
            par64.append(lax.shift_left(iv & 1, 6))

        wrap = [(iota16 + m) & 15 for m in range(16)]
        lat = 8

        @pl.loop(0, 4)
        def _db(db):
            d0 = db * 16
            pend = []
            for m in range(16):
                ddv = wrap[m] + d0
                dblkv = lax.shift_right_logical(ddv, 3)
                dsv = ddv & 7
                for lb in range(8):
                    colv = par64[lb] + ddv
                    val = plsc.load_gather(pair_v.at[bi],
                                           [rowv[lb], colv])
                    pend.append((dblkv, dsv, rowv[lb], val))
                    if len(pend) > lat:
                        a, b, c, v = pend.pop(0)
                        plsc.store_scatter(t_v.at[ti], [a, b, c], v)
            for a, b, c, v in pend:
                plsc.store_scatter(t_v.at[ti], [a, b, c], v)

    def store_tiles(u, ti):
        uu = u0 + u
        pltpu.async_copy(t_v.at[ti],
                         out_hbm.at[uu // _NIB, :, uu % _NIB],
                         osems[ti])

    def wait_store(ti):
        pltpu.make_async_copy(t_v.at[ti], out_hbm.at[0, :, 0],
                              osems[ti]).wait()

    def substep(u, pb, tb, do_ws):
        pb2 = (pb + 2) % _NPB
        pb3 = (pb + 3) % _NPB
        wait_idx(pb2)
        compute_pairidx(pb2)
        fire_gather(pb2)
        idx_dma(u + 3, pb3)
        wait_gather(pb)
        if do_ws:
            wait_store(tb)
        transpose(pb, tb)
        store_tiles(u, tb)

    pltpu.sync_copy(idx_hbm.at[u0 // _NIB, u0 % _NIB], idx_v.at[0])
    u1 = u0 + 1
    pltpu.sync_copy(idx_hbm.at[u1 // _NIB, u1 % _NIB], idx_v.at[1])
    compute_pairidx(0)
    fire_gather(0)
    compute_pairidx(1)
    fire_gather(1)
    idx_dma(2, 2)
    substep(0, 0, 0, False)
    substep(1, 1, 1, False)

    @pl.loop(0, (_UPW - 4) // _NPB)
    def _p(p):
        for s in range(_NPB):
            u = 2 + p * _NPB + s
            substep(u, (2 + s) % _NPB, s % _NTB, True)

    wait_gather(198 % _NPB)
    wait_store(0)
    transpose(198 % _NPB, 0)
    store_tiles(198, 0)
    wait_gather(199 % _NPB)
    wait_store(1)
    transpose(199 % _NPB, 1)
    store_tiles(199, 1)
    wait_idx((199 + 3) % _NPB)
    wait_store(0)
    wait_store(1)


_NCB = 7812


def _bodyA(tT_hbm, tail_hbm, scr_hbm, in_v, t2_v, isems, osems):
    wid = lax.axis_index("s") * _NC + lax.axis_index("c")
    iota16 = lax.iota(jnp.int32, 16)
    iota64 = iota16 * 64
    wrap = [(iota16 + m) & 15 for m in range(16)]
    lvb = [iota16 + lb * 16 for lb in range(8)]
    lvb64 = [iota64 + lb * 1024 for lb in range(8)]

    def fire_in(r, bi):
        cb = jnp.minimum(wid + 32 * r, _NCB - 1)
        start = pl.multiple_of(cb * 128, 128)
        pltpu.async_copy(tT_hbm.at[:, pl.ds(start, 128)], in_v.at[bi],
                         isems[bi])

    def wait_in(bi):
        pltpu.make_async_copy(tT_hbm.at[:, pl.ds(0, 128)], in_v.at[bi],
                              isems[bi]).wait()

    def transpose(bi, ti, nlb):
        @pl.loop(0, 4)
        def _db(db):
            d0 = db * 16
            pend = []
            for m in range(16):
                base_m = wrap[m] + d0
                for lb in range(nlb):
                    ov = lvb64[lb] + base_m
                    val = plsc.load_gather(in_v.at[bi], [base_m, lvb[lb]])
                    i0 = lax.shift_right_logical(ov, 7)
                    i1 = ov & 127
                    pend.append((i0, i1, val))
                    if len(pend) > 8:
                        a, b, v = pend.pop(0)
                        plsc.store_scatter(t2_v.at[ti], [a, b], v)
            for a, b, v in pend:
                plsc.store_scatter(t2_v.at[ti], [a, b], v)

    def store_out(r, ti):
        cb = jnp.minimum(wid + 32 * r, _NCB - 1)
        pltpu.async_copy(t2_v.at[ti], scr_hbm.at[pl.ds(cb * 64, 64)],
                         osems[ti])

    def wait_out(ti):
        pltpu.make_async_copy(t2_v.at[ti], scr_hbm.at[pl.ds(0, 64)],
                              osems[ti]).wait()

    fire_in(0, 0)
    fire_in(1, 1)
    for s in range(2):
        wait_in(s)
        transpose(s, s, 8)
        fire_in(s + 2, s)
        store_out(s, s)

    @pl.loop(0, 121)
    def _r(rr):
        for s in range(2):
            r = 2 + rr * 2 + s
            wait_in(s)
            wait_out(s)
            transpose(s, s, 8)
            fire_in(r + 2, s)
            store_out(r, s)

    wait_in(0)
    wait_out(0)
    transpose(0, 0, 8)
    store_out(244, 0)
    wait_in(1)
    wait_out(0)
    wait_out(1)

    @pl.when(wid == 0)
    def _tail():
        pltpu.sync_copy(tail_hbm, in_v.at[0])
        transpose(0, 0, 4)
        pltpu.sync_copy(t2_v.at[0, pl.ds(0, 32)],
                        scr_hbm.at[pl.ds(_NCB * 64, 32)])


@jax.jit
def _runA(tT, tailT):
    mesh = plsc.VectorSubcoreMesh(core_axis_name="c", subcore_axis_name="s")
    k = pl.kernel(
        _bodyA,
        out_type=jax.ShapeDtypeStruct((500000, 128), jnp.float32),
        mesh=mesh,
        scratch_types=[
            pltpu.VMEM((2, _D, 128), jnp.float32),
            pltpu.VMEM((2, _D, 128), jnp.float32),
            [pltpu.SemaphoreType.DMA] * 2,
            [pltpu.SemaphoreType.DMA] * 2,
        ],
        compiler_params=pltpu.CompilerParams(use_tc_tiling_on_sc=True,
                                             needs_layout_passes=False),
    )
    return k(tT, tailT)


@jax.jit
def _run(idx3, table2):
    mesh = plsc.VectorSubcoreMesh(core_axis_name="c", subcore_axis_name="s")
    k = pl.kernel(
        _body,
        out_type=jax.ShapeDtypeStruct((_NJ, 8, _NIB, 8, 128), jnp.float32),
        mesh=mesh,
        scratch_types=[
            pltpu.VMEM((_NPB, 128), jnp.int32),
            pltpu.VMEM((_NPB, 128), jnp.int32),
            pltpu.VMEM((_NPB, 128, 128), jnp.float32),
            pltpu.VMEM((_NTB, 8, 8, 128), jnp.float32),
        ] + [pltpu.SemaphoreType.DMA] * 9,
        compiler_params=pltpu.CompilerParams(use_tc_tiling_on_sc=True,
                                             needs_layout_passes=False),
    )
    return k(table2, idx3)


def kernel(iid, table):
    tailT = jnp.pad(table[_NCB * 128:], ((0, 64), (0, 0))).T
    table2 = _runA(table.T, tailT)
    idx3 = iid.T.reshape(_NJ, _NIB, 128).astype(jnp.int32)
    out5 = _run(idx3, table2)
    return out5.transpose(2, 4, 0, 1, 3).reshape(_NI, _NJ, _D)
